# Initial kernel scaffold; baseline (speedup 1.0000x reference)
import functools

import jax
import jax.numpy as jnp
from jax import lax
from jax.experimental import pallas as pl
from jax.experimental.pallas import tpu as pltpu
from jax.experimental.pallas import tpu_sc as plsc

_N = 10000
_E = 320000
_D = 128
_NCLS = 47

_NPAD = 10240
_DAUG = 144
_D2 = 64

_NC = 2
_NS = 16
_NW = _NC * _NS
_CHUNK = 128
_CPT = 80
_EPAD = _NW * _CPT * _CHUNK
_STRIPE = _NPAD // _NS


def _make_sc_agg(D):
  mesh = plsc.VectorSubcoreMesh(core_axis_name="c", subcore_axis_name="s")

  @functools.partial(
      pl.kernel,
      out_type=jax.ShapeDtypeStruct((_NC, _NPAD, D), jnp.float32),
      mesh=mesh,
      scratch_types=[
          pltpu.VMEM((_CPT, _CHUNK), jnp.int32),
          pltpu.VMEM((_CPT, _CHUNK), jnp.int32),
          pltpu.VMEM((_CHUNK, D), jnp.float32),
          pltpu.VMEM_SHARED((_NPAD, D), jnp.float32),
          pltpu.SemaphoreType.DMA,
      ],
  )
  def agg(feat_hbm, srcs_hbm, dsts_hbm, zeros_hbm, out_hbm,
          src_v, dst_v, rows_v, acc_sh, sem):
    cid = lax.axis_index("c")
    sid = lax.axis_index("s")
    wid = sid * _NC + cid
    r0 = sid * _STRIPE

    pltpu.sync_copy(zeros_hbm.at[pl.ds(r0, _STRIPE)],
                    acc_sh.at[pl.ds(r0, _STRIPE)])
    pltpu.sync_copy(srcs_hbm.at[wid], src_v)
    pltpu.sync_copy(dsts_hbm.at[wid], dst_v)
    plsc.subcore_barrier()

    def body(c, carry):
      pltpu.async_copy(feat_hbm.at[src_v.at[c]], rows_v, sem).wait()
      pltpu.sync_copy(rows_v, acc_sh.at[dst_v.at[c]], add=True)
      return carry

    lax.fori_loop(0, _CPT, body, 0)

    plsc.subcore_barrier()
    pltpu.sync_copy(acc_sh.at[pl.ds(r0, _STRIPE)],
                    out_hbm.at[cid, pl.ds(r0, _STRIPE)])

  return agg


_agg_aug = _make_sc_agg(_DAUG)
_agg_z = _make_sc_agg(_D2)

_RB = 640
_GRID = _NPAD // _RB


def _tc_mid_body(acca_ref, accb_ref, h_ref, w1l_ref, b1_ref, w1r_ref,
                 w2l_ref, w2r_ref, b2_ref, z_ref, r2_ref, inv_ref):
  acc = acca_ref[...] + accb_ref[...]
  lane = lax.broadcasted_iota(jnp.int32, (1, _DAUG), 1)
  cnt = jnp.sum(jnp.where(lane == _D, acc, 0.0), axis=1, keepdims=True)
  inv = 1.0 / jnp.maximum(cnt, 1.0)
  x1 = (jnp.dot(acc, w1l_ref[...], preferred_element_type=jnp.float32) * inv
        + b1_ref[...]
        + jnp.dot(h_ref[...], w1r_ref[...], preferred_element_type=jnp.float32))
  x1 = jnp.maximum(x1, 0.0)
  z_ref[...] = jnp.dot(x1, w2l_ref[...], preferred_element_type=jnp.float32)
  r2_ref[...] = (jnp.dot(x1, w2r_ref[...], preferred_element_type=jnp.float32)
                 + b2_ref[...])
  inv_ref[...] = jnp.broadcast_to(inv, (_RB, _D2))


def _tc_mid(acc_a, acc_b, h_pad, w1l_aug, b1r, w1r_t, w2l_pad, w2r_pad, b2r):
  blk = lambda r, c: pl.BlockSpec((r, c), lambda i: (i, 0))
  full = lambda r, c: pl.BlockSpec((r, c), lambda i: (0, 0))
  return pl.pallas_call(
      _tc_mid_body,
      grid=(_GRID,),
      in_specs=[
          blk(_RB, _DAUG), blk(_RB, _DAUG), blk(_RB, _D),
          full(_DAUG, _D), full(1, _D), full(_D, _D),
          full(_D, _D2), full(_D, _D2), full(1, _D2),
      ],
      out_specs=[blk(_RB, _D2), blk(_RB, _D2), blk(_RB, _D2)],
      out_shape=[
          jax.ShapeDtypeStruct((_NPAD, _D2), jnp.float32),
          jax.ShapeDtypeStruct((_NPAD, _D2), jnp.float32),
          jax.ShapeDtypeStruct((_NPAD, _D2), jnp.float32),
      ],
  )(acc_a, acc_b, h_pad, w1l_aug, b1r, w1r_t, w2l_pad, w2r_pad, b2r)


def _tc_out_body(acc2a_ref, acc2b_ref, inv_ref, r2_ref, out_ref):
  s = (acc2a_ref[...] + acc2b_ref[...]) * inv_ref[...] + r2_ref[...]
  lane = lax.broadcasted_iota(jnp.int32, (_RB, _D2), 1)
  sm = jnp.where(lane < _NCLS, s, -jnp.inf)
  m = jnp.max(sm, axis=1, keepdims=True)
  e = jnp.where(lane < _NCLS, jnp.exp(sm - m), 0.0)
  out_ref[...] = s - (jnp.log(jnp.sum(e, axis=1, keepdims=True)) + m)


def _tc_out(acc2_a, acc2_b, inv64, r2b):
  blk = pl.BlockSpec((_RB, _D2), lambda i: (i, 0))
  return pl.pallas_call(
      _tc_out_body,
      grid=(_GRID,),
      in_specs=[blk, blk, blk, blk],
      out_specs=blk,
      out_shape=jax.ShapeDtypeStruct((_NPAD, _D2), jnp.float32),
  )(acc2_a, acc2_b, inv64, r2b)


def kernel(h, edge_index, W1l, b1, W1r, W2l, b2, W2r):
  h = h.astype(jnp.float32)
  src = edge_index[0].astype(jnp.int32)
  dst = edge_index[1].astype(jnp.int32)

  pad = _EPAD - _E
  src_p = jnp.concatenate([src, jnp.zeros((pad,), jnp.int32)])
  dst_p = jnp.concatenate([dst, jnp.full((pad,), _N, jnp.int32)])
  src_p = src_p.reshape(_NW, _CPT, _CHUNK)
  dst_p = dst_p.reshape(_NW, _CPT, _CHUNK)

  h_aug = jnp.zeros((_NPAD, _DAUG), jnp.float32)
  h_aug = h_aug.at[:_N, :_D].set(h).at[:_N, _D].set(1.0)
  zeros_aug = jnp.zeros((_NPAD, _DAUG), jnp.float32)
  zeros_z = jnp.zeros((_NPAD, _D2), jnp.float32)
  h_pad = jnp.zeros((_NPAD, _D), jnp.float32).at[:_N].set(h)

  w1l_aug = jnp.zeros((_DAUG, _D), jnp.float32).at[:_D].set(W1l.T)
  w1r_t = W1r.T
  w2l_pad = jnp.zeros((_D, _D2), jnp.float32).at[:, :_NCLS].set(W2l.T)
  w2r_pad = jnp.zeros((_D, _D2), jnp.float32).at[:, :_NCLS].set(W2r.T)
  b1r = b1.reshape(1, _D)
  b2r = jnp.zeros((1, _D2), jnp.float32).at[0, :_NCLS].set(b2)

  acc1 = _agg_aug(h_aug, src_p, dst_p, zeros_aug)
  z, r2b, inv64 = _tc_mid(acc1[0], acc1[1], h_pad, w1l_aug, b1r, w1r_t,
                          w2l_pad, w2r_pad, b2r)
  acc2 = _agg_z(z, src_p, dst_p, zeros_z)
  out = _tc_out(acc2[0], acc2[1], inv64, r2b)
  return out[:_N, :_NCLS]


# speedup vs baseline: 3.7398x; 3.7398x over previous
import functools

import jax
import jax.numpy as jnp
from jax import lax
from jax.experimental import pallas as pl
from jax.experimental.pallas import tpu as pltpu
from jax.experimental.pallas import tpu_sc as plsc

_N = 10000
_E = 320000
_D = 128
_NCLS = 47

_NPAD = 10240
_DAUG = 144
_D2 = 64

_NC = 2
_NS = 16
_NW = _NC * _NS
_CHUNK = 128
_CPT = 80
_EPAD = _NW * _CPT * _CHUNK
_STRIPE = _NPAD // _NS


def _make_sc_agg(D):
  mesh = plsc.VectorSubcoreMesh(core_axis_name="c", subcore_axis_name="s",
                                num_cores=_NC, num_subcores=_NS)

  @functools.partial(
      pl.kernel,
      out_type=jax.ShapeDtypeStruct((_NC, _NPAD, D), jnp.float32),
      mesh=mesh,
      compiler_params=pltpu.CompilerParams(use_tc_tiling_on_sc=False),
      scratch_types=[
          pltpu.VMEM((_CPT, _CHUNK), jnp.int32),
          pltpu.VMEM((_CPT, _CHUNK), jnp.int32),
          pltpu.VMEM((_CHUNK, D), jnp.float32),
          pltpu.VMEM_SHARED((_NPAD, D), jnp.float32),
          pltpu.SemaphoreType.DMA,
      ],
  )
  def agg(feat_hbm, srcs_hbm, dsts_hbm, zeros_hbm, out_hbm,
          src_v, dst_v, rows_v, acc_sh, sem):
    cid = lax.axis_index("c")
    sid = lax.axis_index("s")
    wid = sid * _NC + cid
    r0 = sid * _STRIPE

    pltpu.sync_copy(zeros_hbm.at[pl.ds(r0, _STRIPE)],
                    acc_sh.at[pl.ds(r0, _STRIPE)])
    pltpu.sync_copy(srcs_hbm.at[wid], src_v)
    pltpu.sync_copy(dsts_hbm.at[wid], dst_v)
    plsc.subcore_barrier()

    def body(c, carry):
      pltpu.async_copy(feat_hbm.at[src_v.at[c]], rows_v, sem).wait()
      pltpu.sync_copy(rows_v, acc_sh.at[dst_v.at[c]], add=True)
      return carry

    lax.fori_loop(0, _CPT, body, 0)

    plsc.subcore_barrier()
    pltpu.sync_copy(acc_sh.at[pl.ds(r0, _STRIPE)],
                    out_hbm.at[cid, pl.ds(r0, _STRIPE)])

  return agg


_agg_cache = {}


def _agg_aug(*args):
  if _DAUG not in _agg_cache:
    _agg_cache[_DAUG] = _make_sc_agg(_DAUG)
  return _agg_cache[_DAUG](*args)


def _agg_z(*args):
  if _D2 not in _agg_cache:
    _agg_cache[_D2] = _make_sc_agg(_D2)
  return _agg_cache[_D2](*args)

_RB = 640
_GRID = _NPAD // _RB


def _tc_mid_body(acca_ref, accb_ref, h_ref, w1l_ref, b1_ref, w1r_ref,
                 w2l_ref, w2r_ref, b2_ref, z_ref, r2_ref, inv_ref):
  acc = acca_ref[...] + accb_ref[...]
  lane = lax.broadcasted_iota(jnp.int32, (1, _DAUG), 1)
  cnt = jnp.sum(jnp.where(lane == _D, acc, 0.0), axis=1, keepdims=True)
  inv = 1.0 / jnp.maximum(cnt, 1.0)
  x1 = (jnp.dot(acc, w1l_ref[...], preferred_element_type=jnp.float32) * inv
        + b1_ref[...]
        + jnp.dot(h_ref[...], w1r_ref[...], preferred_element_type=jnp.float32))
  x1 = jnp.maximum(x1, 0.0)
  z_ref[...] = jnp.dot(x1, w2l_ref[...], preferred_element_type=jnp.float32)
  r2_ref[...] = (jnp.dot(x1, w2r_ref[...], preferred_element_type=jnp.float32)
                 + b2_ref[...])
  inv_ref[...] = jnp.broadcast_to(inv, (_RB, _D2))


def _tc_mid(acc_a, acc_b, h_pad, w1l_aug, b1r, w1r_t, w2l_pad, w2r_pad, b2r):
  blk = lambda r, c: pl.BlockSpec((r, c), lambda i: (i, 0))
  full = lambda r, c: pl.BlockSpec((r, c), lambda i: (0, 0))
  return pl.pallas_call(
      _tc_mid_body,
      grid=(_GRID,),
      in_specs=[
          blk(_RB, _DAUG), blk(_RB, _DAUG), blk(_RB, _D),
          full(_DAUG, _D), full(1, _D), full(_D, _D),
          full(_D, _D2), full(_D, _D2), full(1, _D2),
      ],
      out_specs=[blk(_RB, _D2), blk(_RB, _D2), blk(_RB, _D2)],
      out_shape=[
          jax.ShapeDtypeStruct((_NPAD, _D2), jnp.float32),
          jax.ShapeDtypeStruct((_NPAD, _D2), jnp.float32),
          jax.ShapeDtypeStruct((_NPAD, _D2), jnp.float32),
      ],
  )(acc_a, acc_b, h_pad, w1l_aug, b1r, w1r_t, w2l_pad, w2r_pad, b2r)


def _tc_out_body(acc2a_ref, acc2b_ref, inv_ref, r2_ref, out_ref):
  s = (acc2a_ref[...] + acc2b_ref[...]) * inv_ref[...] + r2_ref[...]
  lane = lax.broadcasted_iota(jnp.int32, (_RB, _D2), 1)
  sm = jnp.where(lane < _NCLS, s, -jnp.inf)
  m = jnp.max(sm, axis=1, keepdims=True)
  e = jnp.where(lane < _NCLS, jnp.exp(sm - m), 0.0)
  out_ref[...] = s - (jnp.log(jnp.sum(e, axis=1, keepdims=True)) + m)


def _tc_out(acc2_a, acc2_b, inv64, r2b):
  blk = pl.BlockSpec((_RB, _D2), lambda i: (i, 0))
  return pl.pallas_call(
      _tc_out_body,
      grid=(_GRID,),
      in_specs=[blk, blk, blk, blk],
      out_specs=blk,
      out_shape=jax.ShapeDtypeStruct((_NPAD, _D2), jnp.float32),
  )(acc2_a, acc2_b, inv64, r2b)


def kernel(h, edge_index, W1l, b1, W1r, W2l, b2, W2r):
  h = h.astype(jnp.float32)
  src = edge_index[0].astype(jnp.int32)
  dst = edge_index[1].astype(jnp.int32)

  pad = _EPAD - _E
  src_p = jnp.concatenate([src, jnp.zeros((pad,), jnp.int32)])
  dst_p = jnp.concatenate([dst, jnp.full((pad,), _N, jnp.int32)])
  src_p = src_p.reshape(_NW, _CPT, _CHUNK)
  dst_p = dst_p.reshape(_NW, _CPT, _CHUNK)

  h_aug = jnp.zeros((_NPAD, _DAUG), jnp.float32)
  h_aug = h_aug.at[:_N, :_D].set(h).at[:_N, _D].set(1.0)
  zeros_aug = jnp.zeros((_NPAD, _DAUG), jnp.float32)
  zeros_z = jnp.zeros((_NPAD, _D2), jnp.float32)
  h_pad = jnp.zeros((_NPAD, _D), jnp.float32).at[:_N].set(h)

  w1l_aug = jnp.zeros((_DAUG, _D), jnp.float32).at[:_D].set(W1l.T)
  w1r_t = W1r.T
  w2l_pad = jnp.zeros((_D, _D2), jnp.float32).at[:, :_NCLS].set(W2l.T)
  w2r_pad = jnp.zeros((_D, _D2), jnp.float32).at[:, :_NCLS].set(W2r.T)
  b1r = b1.reshape(1, _D)
  b2r = jnp.zeros((1, _D2), jnp.float32).at[0, :_NCLS].set(b2)

  acc1 = _agg_aug(h_aug, src_p, dst_p, zeros_aug)
  z, r2b, inv64 = _tc_mid(acc1[0], acc1[1], h_pad, w1l_aug, b1r, w1r_t,
                          w2l_pad, w2r_pad, b2r)
  acc2 = _agg_z(z, src_p, dst_p, zeros_z)
  out = _tc_out(acc2[0], acc2[1], inv64, r2b)
  return out[:_N, :_NCLS]


# speedup vs baseline: 4.6907x; 1.2543x over previous
import functools

import jax
import jax.numpy as jnp
from jax import lax
from jax.experimental import pallas as pl
from jax.experimental.pallas import tpu as pltpu
from jax.experimental.pallas import tpu_sc as plsc

_N = 10000
_E = 320000
_D = 128
_NCLS = 47

_NPAD = 10240
_D2 = 48

_NC = 2
_NS = 16
_NW = _NC * _NS
_CHUNK = 128
_CPT = 80
_EPAD = _NW * _CPT * _CHUNK
_STRIPE = _NPAD // _NS
_VPC = _CHUNK // 16
_SCW = 4


def _make_sc_agg(D, with_count):
  mesh = plsc.VectorSubcoreMesh(core_axis_name="c", subcore_axis_name="s",
                                num_cores=_NC, num_subcores=_NS)
  out_type = [jax.ShapeDtypeStruct((_NC, _NPAD, D), jnp.float32)]
  scratch = [
      pltpu.VMEM((_CPT, _CHUNK), jnp.int32),
      pltpu.VMEM((_CPT, _CHUNK), jnp.int32),
      pltpu.VMEM((_CHUNK, D), jnp.float32),
      pltpu.VMEM_SHARED((_NPAD, D), jnp.float32),
      pltpu.SemaphoreType.DMA,
  ]
  if with_count:
    out_type.append(jax.ShapeDtypeStruct((_NW, _NPAD), jnp.float32))
    scratch.append(pltpu.VMEM((_NPAD,), jnp.float32))

  @functools.partial(
      pl.kernel,
      out_type=out_type,
      mesh=mesh,
      compiler_params=pltpu.CompilerParams(use_tc_tiling_on_sc=False,
                                           needs_layout_passes=False),
      scratch_types=scratch,
  )
  def agg(feat_hbm, srcs_hbm, dsts_hbm, zeros_hbm, *rest):
    if with_count:
      out_hbm, cnt_hbm, src_v, dst_v, rows_v, acc_sh, sem, cnt_v = rest
    else:
      out_hbm, src_v, dst_v, rows_v, acc_sh, sem = rest
    cid = lax.axis_index("c")
    sid = lax.axis_index("s")
    wid = sid * _NC + cid
    r0 = sid * _STRIPE

    pltpu.sync_copy(zeros_hbm.at[pl.ds(r0, _STRIPE)],
                    acc_sh.at[pl.ds(r0, _STRIPE)])
    pltpu.sync_copy(srcs_hbm.at[wid], src_v)
    pltpu.sync_copy(dsts_hbm.at[wid], dst_v)
    if with_count:
      def zero_cnt(i, carry):
        cnt_v[pl.ds(i * 16, 16)] = jnp.zeros((16,), jnp.float32)
        return carry
      lax.fori_loop(0, _NPAD // 16, zero_cnt, 0)
    plsc.subcore_barrier()

    def body(c, carry):
      pltpu.async_copy(feat_hbm.at[src_v.at[c]], rows_v, sem).wait()
      if with_count:
        for j in range(_VPC):
          idx = dst_v[c, pl.ds(j * 16, 16)]
          cnts, last = plsc.scan_count(idx)
          plsc.addupdate_scatter(cnt_v, [idx], cnts.astype(jnp.float32),
                                 mask=last)
      pltpu.sync_copy(rows_v, acc_sh.at[dst_v.at[c]], add=True)
      return carry

    lax.fori_loop(0, _CPT, body, 0)

    if with_count:
      pltpu.sync_copy(cnt_v, cnt_hbm.at[wid])
    plsc.subcore_barrier()
    pltpu.sync_copy(acc_sh.at[pl.ds(r0, _STRIPE)],
                    out_hbm.at[cid, pl.ds(r0, _STRIPE)])

  return agg


_agg_cache = {}


def _agg1(*args):
  if 1 not in _agg_cache:
    _agg_cache[1] = _make_sc_agg(_D, True)
  return _agg_cache[1](*args)


def _agg2(*args):
  if 2 not in _agg_cache:
    _agg_cache[2] = _make_sc_agg(_D2, False)
  return _agg_cache[2](*args)


_RB = 640
_GRID = _NPAD // _RB


def _tc_mid_body(acca_ref, accb_ref, cnt_ref, h_ref, w1l_ref, b1_ref,
                 w1r_ref, w2l_ref, w2r_ref, b2_ref, z_ref, r2_ref, inv_ref):
  acc = acca_ref[...] + accb_ref[...]
  ones = jnp.full((_NW, 1), 1.0, dtype=jnp.float32)
  cnt = lax.dot_general(cnt_ref[...], ones, (((0,), (0,)), ((), ())),
                        preferred_element_type=jnp.float32)
  inv = 1.0 / jnp.maximum(cnt, 1.0)
  x1 = (jnp.dot(acc, w1l_ref[...], preferred_element_type=jnp.float32) * inv
        + b1_ref[...]
        + jnp.dot(h_ref[...], w1r_ref[...], preferred_element_type=jnp.float32))
  x1 = jnp.maximum(x1, 0.0)
  z_ref[...] = jnp.dot(x1, w2l_ref[...], preferred_element_type=jnp.float32)
  r2_ref[...] = (jnp.dot(x1, w2r_ref[...], preferred_element_type=jnp.float32)
                 + b2_ref[...])
  inv_ref[...] = jnp.broadcast_to(inv, (_RB, _D2))


def _tc_mid(acc_a, acc_b, cnt_parts, h_pad, w1l_t, b1r, w1r_t,
            w2l_pad, w2r_pad, b2r):
  blk = lambda r, c: pl.BlockSpec((r, c), lambda i: (i, 0))
  full = lambda r, c: pl.BlockSpec((r, c), lambda i: (0, 0))
  return pl.pallas_call(
      _tc_mid_body,
      grid=(_GRID,),
      in_specs=[
          blk(_RB, _D), blk(_RB, _D),
          pl.BlockSpec((_NW, _RB), lambda i: (0, i)),
          blk(_RB, _D),
          full(_D, _D), full(1, _D), full(_D, _D),
          full(_D, _D2), full(_D, _D2), full(1, _D2),
      ],
      out_specs=[blk(_RB, _D2), blk(_RB, _D2), blk(_RB, _D2)],
      out_shape=[
          jax.ShapeDtypeStruct((_NPAD, _D2), jnp.float32),
          jax.ShapeDtypeStruct((_NPAD, _D2), jnp.float32),
          jax.ShapeDtypeStruct((_NPAD, _D2), jnp.float32),
      ],
  )(acc_a, acc_b, cnt_parts, h_pad, w1l_t, b1r, w1r_t, w2l_pad, w2r_pad, b2r)


def _tc_out_body(acc2a_ref, acc2b_ref, inv_ref, r2_ref, out_ref):
  s = (acc2a_ref[...] + acc2b_ref[...]) * inv_ref[...] + r2_ref[...]
  lane = lax.broadcasted_iota(jnp.int32, (_RB, _D2), 1)
  sm = jnp.where(lane < _NCLS, s, -jnp.inf)
  m = jnp.max(sm, axis=1, keepdims=True)
  e = jnp.where(lane < _NCLS, jnp.exp(sm - m), 0.0)
  out_ref[...] = s - (jnp.log(jnp.sum(e, axis=1, keepdims=True)) + m)


def _tc_out(acc2_a, acc2_b, inv48, r2b):
  blk = pl.BlockSpec((_RB, _D2), lambda i: (i, 0))
  return pl.pallas_call(
      _tc_out_body,
      grid=(_GRID,),
      in_specs=[blk, blk, blk, blk],
      out_specs=blk,
      out_shape=jax.ShapeDtypeStruct((_NPAD, _D2), jnp.float32),
  )(acc2_a, acc2_b, inv48, r2b)


def kernel(h, edge_index, W1l, b1, W1r, W2l, b2, W2r):
  h = h.astype(jnp.float32)
  src = edge_index[0].astype(jnp.int32)
  dst = edge_index[1].astype(jnp.int32)

  pad = _EPAD - _E
  src_p = jnp.concatenate([src, jnp.zeros((pad,), jnp.int32)])
  dst_p = jnp.concatenate([dst, jnp.full((pad,), _N, jnp.int32)])
  src_p = src_p.reshape(_NW, _CPT, _CHUNK)
  dst_p = dst_p.reshape(_NW, _CPT, _CHUNK)

  h_pad = jnp.zeros((_NPAD, _D), jnp.float32).at[:_N].set(h)
  zeros_d = jnp.zeros((_NPAD, _D), jnp.float32)
  zeros_z = jnp.zeros((_NPAD, _D2), jnp.float32)

  w1l_t = W1l.T
  w1r_t = W1r.T
  w2l_pad = jnp.zeros((_D, _D2), jnp.float32).at[:, :_NCLS].set(W2l.T)
  w2r_pad = jnp.zeros((_D, _D2), jnp.float32).at[:, :_NCLS].set(W2r.T)
  b1r = b1.reshape(1, _D)
  b2r = jnp.zeros((1, _D2), jnp.float32).at[0, :_NCLS].set(b2)

  acc1, cnt_parts = _agg1(h_pad, src_p, dst_p, zeros_d)
  z, r2b, inv48 = _tc_mid(acc1[0], acc1[1], cnt_parts, h_pad, w1l_t, b1r,
                          w1r_t, w2l_pad, w2r_pad, b2r)
  acc2 = _agg2(z, src_p, dst_p, zeros_z)[0]
  out = _tc_out(acc2[0], acc2[1], inv48, r2b)
  return out[:_N, :_NCLS]


# speedup vs baseline: 4.7321x; 1.0088x over previous
import functools

import jax
import jax.numpy as jnp
from jax import lax
from jax.experimental import pallas as pl
from jax.experimental.pallas import tpu as pltpu
from jax.experimental.pallas import tpu_sc as plsc

_N = 10000
_E = 320000
_D = 128
_NCLS = 47

_NPAD = 10240
_D2 = 48

_NC = 2
_NS = 16
_NW = _NC * _NS
_CHUNK = 128
_CPT = 80
_EPAD = _NW * _CPT * _CHUNK
_STRIPE = _NPAD // _NS
_VPC = _CHUNK // 16
_SCW = 4


def _make_sc_agg(D, with_count):
  mesh = plsc.VectorSubcoreMesh(core_axis_name="c", subcore_axis_name="s",
                                num_cores=_NC, num_subcores=_NS)
  out_type = [jax.ShapeDtypeStruct((_NC, _NPAD, D), jnp.float32)]
  scratch = [
      pltpu.VMEM((_CPT, _CHUNK), jnp.int32),
      pltpu.VMEM((_CPT, _CHUNK), jnp.int32),
      pltpu.VMEM((_CHUNK, D), jnp.float32),
      pltpu.VMEM_SHARED((_NPAD, D), jnp.float32),
      pltpu.SemaphoreType.DMA,
  ]
  if with_count:
    out_type.append(jax.ShapeDtypeStruct((_NW, _NPAD), jnp.float32))
    scratch.append(pltpu.VMEM((_NPAD,), jnp.float32))

  @functools.partial(
      pl.kernel,
      out_type=out_type,
      mesh=mesh,
      compiler_params=pltpu.CompilerParams(use_tc_tiling_on_sc=False,
                                           needs_layout_passes=False),
      scratch_types=scratch,
  )
  def agg(feat_hbm, srcs_hbm, dsts_hbm, zeros_hbm, *rest):
    if with_count:
      out_hbm, cnt_hbm, src_v, dst_v, rows_v, acc_sh, sem, cnt_v = rest
    else:
      out_hbm, src_v, dst_v, rows_v, acc_sh, sem = rest
    cid = lax.axis_index("c")
    sid = lax.axis_index("s")
    wid = sid * _NC + cid
    r0 = sid * _STRIPE

    pltpu.sync_copy(zeros_hbm.at[pl.ds(r0, _STRIPE)],
                    acc_sh.at[pl.ds(r0, _STRIPE)])
    pltpu.sync_copy(srcs_hbm.at[wid], src_v)
    pltpu.sync_copy(dsts_hbm.at[wid], dst_v)
    if with_count:
      def zero_cnt(i, carry):
        cnt_v[pl.ds(i * 16, 16)] = jnp.zeros((16,), jnp.float32)
        return carry
      lax.fori_loop(0, _NPAD // 16, zero_cnt, 0)
    plsc.subcore_barrier()

    def body(c, carry):
      gat = pltpu.async_copy(feat_hbm.at[src_v.at[c]], rows_v, sem)
      if with_count:
        for j in range(_VPC):
          idx = dst_v[c, pl.ds(j * 16, 16)]
          cnts, last = plsc.scan_count(idx)
          plsc.addupdate_scatter(cnt_v, [idx], cnts.astype(jnp.float32),
                                 mask=last)
      gat.wait()
      pltpu.sync_copy(rows_v, acc_sh.at[dst_v.at[c]], add=True)
      return carry

    lax.fori_loop(0, _CPT, body, 0)

    if with_count:
      pltpu.sync_copy(cnt_v, cnt_hbm.at[wid])
    plsc.subcore_barrier()
    pltpu.sync_copy(acc_sh.at[pl.ds(r0, _STRIPE)],
                    out_hbm.at[cid, pl.ds(r0, _STRIPE)])

  return agg


_agg_cache = {}


def _agg1(*args):
  if 1 not in _agg_cache:
    _agg_cache[1] = _make_sc_agg(_D, True)
  return _agg_cache[1](*args)


def _agg2(*args):
  if 2 not in _agg_cache:
    _agg_cache[2] = _make_sc_agg(_D2, False)
  return _agg_cache[2](*args)


_RB = 640
_GRID = _NPAD // _RB


def _tc_mid_body(acca_ref, accb_ref, cnt_ref, h_ref, w1l_ref, b1_ref,
                 w1r_ref, w2l_ref, w2r_ref, b2_ref, z_ref, r2_ref, inv_ref):
  acc = acca_ref[...] + accb_ref[...]
  ones = jnp.full((_NW, 1), 1.0, dtype=jnp.float32)
  cnt = lax.dot_general(cnt_ref[...], ones, (((0,), (0,)), ((), ())),
                        preferred_element_type=jnp.float32)
  inv = 1.0 / jnp.maximum(cnt, 1.0)
  x1 = (jnp.dot(acc, w1l_ref[...], preferred_element_type=jnp.float32) * inv
        + b1_ref[...]
        + jnp.dot(h_ref[...], w1r_ref[...], preferred_element_type=jnp.float32))
  x1 = jnp.maximum(x1, 0.0)
  z_ref[...] = jnp.dot(x1, w2l_ref[...], preferred_element_type=jnp.float32)
  r2_ref[...] = (jnp.dot(x1, w2r_ref[...], preferred_element_type=jnp.float32)
                 + b2_ref[...])
  inv_ref[...] = jnp.broadcast_to(inv, (_RB, _D2))


def _tc_mid(acc_a, acc_b, cnt_parts, h_pad, w1l_t, b1r, w1r_t,
            w2l_pad, w2r_pad, b2r):
  blk = lambda r, c: pl.BlockSpec((r, c), lambda i: (i, 0))
  full = lambda r, c: pl.BlockSpec((r, c), lambda i: (0, 0))
  return pl.pallas_call(
      _tc_mid_body,
      grid=(_GRID,),
      in_specs=[
          blk(_RB, _D), blk(_RB, _D),
          pl.BlockSpec((_NW, _RB), lambda i: (0, i)),
          blk(_RB, _D),
          full(_D, _D), full(1, _D), full(_D, _D),
          full(_D, _D2), full(_D, _D2), full(1, _D2),
      ],
      out_specs=[blk(_RB, _D2), blk(_RB, _D2), blk(_RB, _D2)],
      out_shape=[
          jax.ShapeDtypeStruct((_NPAD, _D2), jnp.float32),
          jax.ShapeDtypeStruct((_NPAD, _D2), jnp.float32),
          jax.ShapeDtypeStruct((_NPAD, _D2), jnp.float32),
      ],
  )(acc_a, acc_b, cnt_parts, h_pad, w1l_t, b1r, w1r_t, w2l_pad, w2r_pad, b2r)


def _tc_out_body(acc2a_ref, acc2b_ref, inv_ref, r2_ref, out_ref):
  s = (acc2a_ref[...] + acc2b_ref[...]) * inv_ref[...] + r2_ref[...]
  lane = lax.broadcasted_iota(jnp.int32, (_RB, _D2), 1)
  sm = jnp.where(lane < _NCLS, s, -jnp.inf)
  m = jnp.max(sm, axis=1, keepdims=True)
  e = jnp.where(lane < _NCLS, jnp.exp(sm - m), 0.0)
  out_ref[...] = s - (jnp.log(jnp.sum(e, axis=1, keepdims=True)) + m)


def _tc_out(acc2_a, acc2_b, inv48, r2b):
  blk = pl.BlockSpec((_RB, _D2), lambda i: (i, 0))
  return pl.pallas_call(
      _tc_out_body,
      grid=(_GRID,),
      in_specs=[blk, blk, blk, blk],
      out_specs=blk,
      out_shape=jax.ShapeDtypeStruct((_NPAD, _D2), jnp.float32),
  )(acc2_a, acc2_b, inv48, r2b)


def kernel(h, edge_index, W1l, b1, W1r, W2l, b2, W2r):
  h = h.astype(jnp.float32)
  src = edge_index[0].astype(jnp.int32)
  dst = edge_index[1].astype(jnp.int32)

  pad = _EPAD - _E
  src_p = jnp.concatenate([src, jnp.zeros((pad,), jnp.int32)])
  dst_p = jnp.concatenate([dst, jnp.full((pad,), _N, jnp.int32)])
  src_p = src_p.reshape(_NW, _CPT, _CHUNK)
  dst_p = dst_p.reshape(_NW, _CPT, _CHUNK)

  h_pad = jnp.zeros((_NPAD, _D), jnp.float32).at[:_N].set(h)
  zeros_d = jnp.zeros((_NPAD, _D), jnp.float32)
  zeros_z = jnp.zeros((_NPAD, _D2), jnp.float32)

  w1l_t = W1l.T
  w1r_t = W1r.T
  w2l_pad = jnp.zeros((_D, _D2), jnp.float32).at[:, :_NCLS].set(W2l.T)
  w2r_pad = jnp.zeros((_D, _D2), jnp.float32).at[:, :_NCLS].set(W2r.T)
  b1r = b1.reshape(1, _D)
  b2r = jnp.zeros((1, _D2), jnp.float32).at[0, :_NCLS].set(b2)

  acc1, cnt_parts = _agg1(h_pad, src_p, dst_p, zeros_d)
  z, r2b, inv48 = _tc_mid(acc1[0], acc1[1], cnt_parts, h_pad, w1l_t, b1r,
                          w1r_t, w2l_pad, w2r_pad, b2r)
  acc2 = _agg2(z, src_p, dst_p, zeros_z)[0]
  out = _tc_out(acc2[0], acc2[1], inv48, r2b)
  return out[:_N, :_NCLS]


# speedup vs baseline: 4.7614x; 1.0062x over previous
import functools

import jax
import jax.numpy as jnp
from jax import lax
from jax.experimental import pallas as pl
from jax.experimental.pallas import tpu as pltpu
from jax.experimental.pallas import tpu_sc as plsc

_N = 10000
_E = 320000
_D = 128
_NCLS = 47

_NPAD = 10240
_D2 = 48

_NC = 2
_NS = 16
_NW = _NC * _NS
_CHUNK = 128
_CPT = 80
_EPAD = _NW * _CPT * _CHUNK
_STRIPE = _NPAD // _NS
_VPC = _CHUNK // 16
_SCW = 4


def _make_sc_agg(D, with_count):
  mesh = plsc.VectorSubcoreMesh(core_axis_name="c", subcore_axis_name="s",
                                num_cores=_NC, num_subcores=_NS)
  out_type = [jax.ShapeDtypeStruct((_NC, _NPAD, D), jnp.float32)]
  scratch = [
      pltpu.VMEM((_CPT, _CHUNK), jnp.int32),
      pltpu.VMEM((_CPT, _CHUNK), jnp.int32),
      pltpu.VMEM((_CHUNK, D), jnp.float32),
      pltpu.VMEM_SHARED((_NPAD, D), jnp.float32),
      pltpu.SemaphoreType.DMA,
  ]
  if with_count:
    out_type.append(jax.ShapeDtypeStruct((_NW, _NPAD), jnp.float32))
    scratch.append(pltpu.VMEM((_NPAD,), jnp.float32))

  @functools.partial(
      pl.kernel,
      out_type=out_type,
      mesh=mesh,
      compiler_params=pltpu.CompilerParams(use_tc_tiling_on_sc=False,
                                           needs_layout_passes=False),
      scratch_types=scratch,
  )
  def agg(feat_hbm, srcs_hbm, dsts_hbm, *rest):
    if with_count:
      out_hbm, cnt_hbm, src_v, dst_v, rows_v, acc_sh, sem, cnt_v = rest
    else:
      out_hbm, src_v, dst_v, rows_v, acc_sh, sem = rest
    cid = lax.axis_index("c")
    sid = lax.axis_index("s")
    wid = sid * _NC + cid
    r0 = sid * _STRIPE

    pltpu.sync_copy(srcs_hbm.at[wid], src_v)
    pltpu.sync_copy(dsts_hbm.at[wid], dst_v)

    def zero_rows(i, carry):
      def zero_lane(j, carry2):
        rows_v[i, pl.ds(j * 16, 16)] = jnp.zeros((16,), jnp.float32)
        return carry2
      return lax.fori_loop(0, D // 16, zero_lane, carry)
    lax.fori_loop(0, _CHUNK, zero_rows, 0)

    def zero_stripe(k, carry):
      pltpu.sync_copy(rows_v, acc_sh.at[pl.ds(r0 + k * _CHUNK, _CHUNK)])
      return carry
    lax.fori_loop(0, _STRIPE // _CHUNK, zero_stripe, 0)
    if with_count:
      def zero_cnt(i, carry):
        cnt_v[pl.ds(i * 16, 16)] = jnp.zeros((16,), jnp.float32)
        return carry
      lax.fori_loop(0, _NPAD // 16, zero_cnt, 0)
    plsc.subcore_barrier()

    def body(c, carry):
      gat = pltpu.async_copy(feat_hbm.at[src_v.at[c]], rows_v, sem)
      if with_count:
        for j in range(_VPC):
          idx = dst_v[c, pl.ds(j * 16, 16)]
          cnts, last = plsc.scan_count(idx)
          plsc.addupdate_scatter(cnt_v, [idx], cnts.astype(jnp.float32),
                                 mask=last)
      gat.wait()
      pltpu.sync_copy(rows_v, acc_sh.at[dst_v.at[c]], add=True)
      return carry

    lax.fori_loop(0, _CPT, body, 0)

    if with_count:
      pltpu.sync_copy(cnt_v, cnt_hbm.at[wid])
    plsc.subcore_barrier()
    pltpu.sync_copy(acc_sh.at[pl.ds(r0, _STRIPE)],
                    out_hbm.at[cid, pl.ds(r0, _STRIPE)])

  return agg


_agg_cache = {}


def _agg1(*args):
  if 1 not in _agg_cache:
    _agg_cache[1] = _make_sc_agg(_D, True)
  return _agg_cache[1](*args)


def _agg2(*args):
  if 2 not in _agg_cache:
    _agg_cache[2] = _make_sc_agg(_D2, False)
  return _agg_cache[2](*args)


_RB = 640
_GRID = _NPAD // _RB


def _tc_mid_body(acca_ref, accb_ref, cnt_ref, h_ref, w1l_ref, b1_ref,
                 w1r_ref, w2l_ref, w2r_ref, b2_ref, z_ref, r2_ref, inv_ref):
  acc = acca_ref[...] + accb_ref[...]
  ones = jnp.full((_NW, 1), 1.0, dtype=jnp.float32)
  cnt = lax.dot_general(cnt_ref[...], ones, (((0,), (0,)), ((), ())),
                        preferred_element_type=jnp.float32)
  inv = 1.0 / jnp.maximum(cnt, 1.0)
  x1 = (jnp.dot(acc, w1l_ref[...], preferred_element_type=jnp.float32) * inv
        + b1_ref[...]
        + jnp.dot(h_ref[...], w1r_ref[...], preferred_element_type=jnp.float32))
  x1 = jnp.maximum(x1, 0.0)
  z_ref[...] = jnp.dot(x1, w2l_ref[...], preferred_element_type=jnp.float32)
  r2_ref[...] = (jnp.dot(x1, w2r_ref[...], preferred_element_type=jnp.float32)
                 + b2_ref[...])
  inv_ref[...] = jnp.broadcast_to(inv, (_RB, _D2))


def _tc_mid(acc_a, acc_b, cnt_parts, h_pad, w1l_t, b1r, w1r_t,
            w2l_pad, w2r_pad, b2r):
  blk = lambda r, c: pl.BlockSpec((r, c), lambda i: (i, 0))
  full = lambda r, c: pl.BlockSpec((r, c), lambda i: (0, 0))
  return pl.pallas_call(
      _tc_mid_body,
      grid=(_GRID,),
      in_specs=[
          blk(_RB, _D), blk(_RB, _D),
          pl.BlockSpec((_NW, _RB), lambda i: (0, i)),
          blk(_RB, _D),
          full(_D, _D), full(1, _D), full(_D, _D),
          full(_D, _D2), full(_D, _D2), full(1, _D2),
      ],
      out_specs=[blk(_RB, _D2), blk(_RB, _D2), blk(_RB, _D2)],
      out_shape=[
          jax.ShapeDtypeStruct((_NPAD, _D2), jnp.float32),
          jax.ShapeDtypeStruct((_NPAD, _D2), jnp.float32),
          jax.ShapeDtypeStruct((_NPAD, _D2), jnp.float32),
      ],
  )(acc_a, acc_b, cnt_parts, h_pad, w1l_t, b1r, w1r_t, w2l_pad, w2r_pad, b2r)


def _tc_out_body(acc2a_ref, acc2b_ref, inv_ref, r2_ref, out_ref):
  s = (acc2a_ref[...] + acc2b_ref[...]) * inv_ref[...] + r2_ref[...]
  lane = lax.broadcasted_iota(jnp.int32, (_RB, _D2), 1)
  sm = jnp.where(lane < _NCLS, s, -jnp.inf)
  m = jnp.max(sm, axis=1, keepdims=True)
  e = jnp.where(lane < _NCLS, jnp.exp(sm - m), 0.0)
  out_ref[...] = s - (jnp.log(jnp.sum(e, axis=1, keepdims=True)) + m)


def _tc_out(acc2_a, acc2_b, inv48, r2b):
  blk = pl.BlockSpec((_RB, _D2), lambda i: (i, 0))
  return pl.pallas_call(
      _tc_out_body,
      grid=(_GRID,),
      in_specs=[blk, blk, blk, blk],
      out_specs=blk,
      out_shape=jax.ShapeDtypeStruct((_NPAD, _D2), jnp.float32),
  )(acc2_a, acc2_b, inv48, r2b)


def kernel(h, edge_index, W1l, b1, W1r, W2l, b2, W2r):
  h = h.astype(jnp.float32)
  src = edge_index[0].astype(jnp.int32)
  dst = edge_index[1].astype(jnp.int32)

  pad = _EPAD - _E
  src_p = jnp.concatenate([src, jnp.zeros((pad,), jnp.int32)])
  dst_p = jnp.concatenate([dst, jnp.full((pad,), _N, jnp.int32)])
  src_p = src_p.reshape(_NW, _CPT, _CHUNK)
  dst_p = dst_p.reshape(_NW, _CPT, _CHUNK)

  h_pad = jnp.zeros((_NPAD, _D), jnp.float32).at[:_N].set(h)

  w1l_t = W1l.T
  w1r_t = W1r.T
  w2l_pad = jnp.zeros((_D, _D2), jnp.float32).at[:, :_NCLS].set(W2l.T)
  w2r_pad = jnp.zeros((_D, _D2), jnp.float32).at[:, :_NCLS].set(W2r.T)
  b1r = b1.reshape(1, _D)
  b2r = jnp.zeros((1, _D2), jnp.float32).at[0, :_NCLS].set(b2)

  acc1, cnt_parts = _agg1(h_pad, src_p, dst_p)
  z, r2b, inv48 = _tc_mid(acc1[0], acc1[1], cnt_parts, h_pad, w1l_t, b1r,
                          w1r_t, w2l_pad, w2r_pad, b2r)
  acc2 = _agg2(z, src_p, dst_p)[0]
  out = _tc_out(acc2[0], acc2[1], inv48, r2b)
  return out[:_N, :_NCLS]


# speedup vs baseline: 5.6379x; 1.1841x over previous
import functools

import jax
import jax.numpy as jnp
from jax import lax
from jax.experimental import pallas as pl
from jax.experimental.pallas import tpu as pltpu
from jax.experimental.pallas import tpu_sc as plsc

_N = 10000
_E = 320000
_D = 128
_NCLS = 47

_NPAD = 10240
_D2 = 48

_NC = 2
_NS = 16
_NW = _NC * _NS
_CHUNK = 128
_CPT = 80
_EPAD = _NW * _CPT * _CHUNK
_STRIPE = _NPAD // _NS
_VPC = _CHUNK // 16
_SCW = 4


def _make_sc_agg(D, with_count, feat_in_spmem=False):
  mesh = plsc.VectorSubcoreMesh(core_axis_name="c", subcore_axis_name="s",
                                num_cores=_NC, num_subcores=_NS)
  out_type = [jax.ShapeDtypeStruct((_NC, _NPAD, D), jnp.float32)]
  scratch = [
      pltpu.VMEM((_CPT, _CHUNK), jnp.int32),
      pltpu.VMEM((_CPT, _CHUNK), jnp.int32),
      pltpu.VMEM((_CHUNK, D), jnp.float32),
      pltpu.VMEM_SHARED((_NPAD, D), jnp.float32),
      pltpu.SemaphoreType.DMA,
  ]
  if with_count:
    out_type.append(jax.ShapeDtypeStruct((_NW, _NPAD), jnp.float32))
    scratch.append(pltpu.VMEM((_NPAD,), jnp.float32))
  if feat_in_spmem:
    scratch.append(pltpu.VMEM_SHARED((_NPAD, D), jnp.float32))

  @functools.partial(
      pl.kernel,
      out_type=out_type,
      mesh=mesh,
      compiler_params=pltpu.CompilerParams(use_tc_tiling_on_sc=False,
                                           needs_layout_passes=False),
      scratch_types=scratch,
  )
  def agg(feat_hbm, srcs_hbm, dsts_hbm, *rest):
    feat_sh = rest[-1] if feat_in_spmem else None
    if feat_in_spmem:
      rest = rest[:-1]
    if with_count:
      out_hbm, cnt_hbm, src_v, dst_v, rows_v, acc_sh, sem, cnt_v = rest
    else:
      out_hbm, src_v, dst_v, rows_v, acc_sh, sem = rest
    cid = lax.axis_index("c")
    sid = lax.axis_index("s")
    wid = sid * _NC + cid
    r0 = sid * _STRIPE

    pltpu.sync_copy(srcs_hbm.at[wid], src_v)
    pltpu.sync_copy(dsts_hbm.at[wid], dst_v)

    def zero_rows(i, carry):
      def zero_lane(j, carry2):
        rows_v[i, pl.ds(j * 16, 16)] = jnp.zeros((16,), jnp.float32)
        return carry2
      return lax.fori_loop(0, D // 16, zero_lane, carry)
    lax.fori_loop(0, _CHUNK, zero_rows, 0)

    def zero_stripe(k, carry):
      pltpu.sync_copy(rows_v, acc_sh.at[pl.ds(r0 + k * _CHUNK, _CHUNK)])
      return carry
    lax.fori_loop(0, _STRIPE // _CHUNK, zero_stripe, 0)
    if feat_in_spmem:
      pltpu.sync_copy(feat_hbm.at[pl.ds(r0, _STRIPE)],
                      feat_sh.at[pl.ds(r0, _STRIPE)])
    if with_count:
      def zero_cnt(i, carry):
        cnt_v[pl.ds(i * 16, 16)] = jnp.zeros((16,), jnp.float32)
        return carry
      lax.fori_loop(0, _NPAD // 16, zero_cnt, 0)
    plsc.subcore_barrier()

    def body(c, carry):
      feat_ref = feat_sh if feat_in_spmem else feat_hbm
      gat = pltpu.async_copy(feat_ref.at[src_v.at[c]], rows_v, sem)
      if with_count:
        for j in range(_VPC):
          idx = dst_v[c, pl.ds(j * 16, 16)]
          cnts, last = plsc.scan_count(idx)
          plsc.addupdate_scatter(cnt_v, [idx], cnts.astype(jnp.float32),
                                 mask=last)
      gat.wait()
      pltpu.sync_copy(rows_v, acc_sh.at[dst_v.at[c]], add=True)
      return carry

    lax.fori_loop(0, _CPT, body, 0)

    if with_count:
      pltpu.sync_copy(cnt_v, cnt_hbm.at[wid])
    plsc.subcore_barrier()
    pltpu.sync_copy(acc_sh.at[pl.ds(r0, _STRIPE)],
                    out_hbm.at[cid, pl.ds(r0, _STRIPE)])

  return agg


_agg_cache = {}


def _agg1(*args):
  if 1 not in _agg_cache:
    _agg_cache[1] = _make_sc_agg(_D, True)
  return _agg_cache[1](*args)


def _agg2(*args):
  if 2 not in _agg_cache:
    _agg_cache[2] = _make_sc_agg(_D2, False, feat_in_spmem=True)
  return _agg_cache[2](*args)


_RB = 640
_GRID = _NPAD // _RB


def _tc_mid_body(acca_ref, accb_ref, cnt_ref, h_ref, w1l_ref, b1_ref,
                 w1r_ref, w2l_ref, w2r_ref, b2_ref, z_ref, r2_ref, inv_ref):
  acc = acca_ref[...] + accb_ref[...]
  ones = jnp.full((_NW, 1), 1.0, dtype=jnp.float32)
  cnt = lax.dot_general(cnt_ref[...], ones, (((0,), (0,)), ((), ())),
                        preferred_element_type=jnp.float32)
  inv = 1.0 / jnp.maximum(cnt, 1.0)
  x1 = (jnp.dot(acc, w1l_ref[...], preferred_element_type=jnp.float32) * inv
        + b1_ref[...]
        + jnp.dot(h_ref[...], w1r_ref[...], preferred_element_type=jnp.float32))
  x1 = jnp.maximum(x1, 0.0)
  z_ref[...] = jnp.dot(x1, w2l_ref[...], preferred_element_type=jnp.float32)
  r2_ref[...] = (jnp.dot(x1, w2r_ref[...], preferred_element_type=jnp.float32)
                 + b2_ref[...])
  inv_ref[...] = jnp.broadcast_to(inv, (_RB, _D2))


def _tc_mid(acc_a, acc_b, cnt_parts, h_pad, w1l_t, b1r, w1r_t,
            w2l_pad, w2r_pad, b2r):
  blk = lambda r, c: pl.BlockSpec((r, c), lambda i: (i, 0))
  full = lambda r, c: pl.BlockSpec((r, c), lambda i: (0, 0))
  return pl.pallas_call(
      _tc_mid_body,
      grid=(_GRID,),
      in_specs=[
          blk(_RB, _D), blk(_RB, _D),
          pl.BlockSpec((_NW, _RB), lambda i: (0, i)),
          blk(_RB, _D),
          full(_D, _D), full(1, _D), full(_D, _D),
          full(_D, _D2), full(_D, _D2), full(1, _D2),
      ],
      out_specs=[blk(_RB, _D2), blk(_RB, _D2), blk(_RB, _D2)],
      out_shape=[
          jax.ShapeDtypeStruct((_NPAD, _D2), jnp.float32),
          jax.ShapeDtypeStruct((_NPAD, _D2), jnp.float32),
          jax.ShapeDtypeStruct((_NPAD, _D2), jnp.float32),
      ],
  )(acc_a, acc_b, cnt_parts, h_pad, w1l_t, b1r, w1r_t, w2l_pad, w2r_pad, b2r)


def _tc_out_body(acc2a_ref, acc2b_ref, inv_ref, r2_ref, out_ref):
  s = (acc2a_ref[...] + acc2b_ref[...]) * inv_ref[...] + r2_ref[...]
  lane = lax.broadcasted_iota(jnp.int32, (_RB, _D2), 1)
  sm = jnp.where(lane < _NCLS, s, -jnp.inf)
  m = jnp.max(sm, axis=1, keepdims=True)
  e = jnp.where(lane < _NCLS, jnp.exp(sm - m), 0.0)
  out_ref[...] = s - (jnp.log(jnp.sum(e, axis=1, keepdims=True)) + m)


def _tc_out(acc2_a, acc2_b, inv48, r2b):
  blk = pl.BlockSpec((_RB, _D2), lambda i: (i, 0))
  return pl.pallas_call(
      _tc_out_body,
      grid=(_GRID,),
      in_specs=[blk, blk, blk, blk],
      out_specs=blk,
      out_shape=jax.ShapeDtypeStruct((_NPAD, _D2), jnp.float32),
  )(acc2_a, acc2_b, inv48, r2b)


def kernel(h, edge_index, W1l, b1, W1r, W2l, b2, W2r):
  h = h.astype(jnp.float32)
  src = edge_index[0].astype(jnp.int32)
  dst = edge_index[1].astype(jnp.int32)

  pad = _EPAD - _E
  src_p = jnp.concatenate([src, jnp.zeros((pad,), jnp.int32)])
  dst_p = jnp.concatenate([dst, jnp.full((pad,), _N, jnp.int32)])
  src_p = src_p.reshape(_NW, _CPT, _CHUNK)
  dst_p = dst_p.reshape(_NW, _CPT, _CHUNK)

  h_pad = jnp.zeros((_NPAD, _D), jnp.float32).at[:_N].set(h)

  w1l_t = W1l.T
  w1r_t = W1r.T
  w2l_pad = jnp.zeros((_D, _D2), jnp.float32).at[:, :_NCLS].set(W2l.T)
  w2r_pad = jnp.zeros((_D, _D2), jnp.float32).at[:, :_NCLS].set(W2r.T)
  b1r = b1.reshape(1, _D)
  b2r = jnp.zeros((1, _D2), jnp.float32).at[0, :_NCLS].set(b2)

  acc1, cnt_parts = _agg1(h_pad, src_p, dst_p)
  z, r2b, inv48 = _tc_mid(acc1[0], acc1[1], cnt_parts, h_pad, w1l_t, b1r,
                          w1r_t, w2l_pad, w2r_pad, b2r)
  acc2 = _agg2(z, src_p, dst_p)[0]
  out = _tc_out(acc2[0], acc2[1], inv48, r2b)
  return out[:_N, :_NCLS]


# speedup vs baseline: 8.5555x; 1.5175x over previous
import functools

import jax
import jax.numpy as jnp
from jax import lax
from jax.experimental import pallas as pl
from jax.experimental.pallas import tpu as pltpu
from jax.experimental.pallas import tpu_sc as plsc

_N = 10000
_E = 320000
_D = 128
_NCLS = 47

_NPAD = 10240
_D2 = 48
_DH = 64

_NC = 2
_NS = 16
_NW = _NC * _NS
_CHUNK = 128
_CPT = 80
_EPAD = _NW * _CPT * _CHUNK
_STRIPE = _NPAD // _NS
_VPC = _CHUNK // 16
_SCW = 4


def _make_sc_agg(D, with_count, feat_in_spmem=False):
  mesh = plsc.VectorSubcoreMesh(core_axis_name="c", subcore_axis_name="s",
                                num_cores=_NC, num_subcores=_NS)
  out_type = [jax.ShapeDtypeStruct((_NC, _NPAD, D), jnp.float32)]
  scratch = [
      pltpu.VMEM((_CPT, _CHUNK), jnp.int32),
      pltpu.VMEM((_CPT, _CHUNK), jnp.int32),
      pltpu.VMEM((_CHUNK, D), jnp.float32),
      pltpu.VMEM_SHARED((_NPAD, D), jnp.float32),
      pltpu.SemaphoreType.DMA,
  ]
  if with_count:
    out_type.append(jax.ShapeDtypeStruct((_NW, _NPAD), jnp.float32))
    scratch.append(pltpu.VMEM((_NPAD,), jnp.float32))
  if feat_in_spmem:
    scratch.append(pltpu.VMEM_SHARED((_NPAD, D), jnp.float32))

  @functools.partial(
      pl.kernel,
      out_type=out_type,
      mesh=mesh,
      compiler_params=pltpu.CompilerParams(use_tc_tiling_on_sc=False,
                                           needs_layout_passes=False),
      scratch_types=scratch,
  )
  def agg(feat_hbm, srcs_hbm, dsts_hbm, *rest):
    feat_sh = rest[-1] if feat_in_spmem else None
    if feat_in_spmem:
      rest = rest[:-1]
    if with_count:
      out_hbm, cnt_hbm, src_v, dst_v, rows_v, acc_sh, sem, cnt_v = rest
    else:
      out_hbm, src_v, dst_v, rows_v, acc_sh, sem = rest
    cid = lax.axis_index("c")
    sid = lax.axis_index("s")
    wid = sid * _NC + cid
    r0 = sid * _STRIPE

    pltpu.sync_copy(srcs_hbm.at[wid], src_v)
    pltpu.sync_copy(dsts_hbm.at[wid], dst_v)

    def zero_rows(i, carry):
      def zero_lane(j, carry2):
        rows_v[i, pl.ds(j * 16, 16)] = jnp.zeros((16,), jnp.float32)
        return carry2
      return lax.fori_loop(0, D // 16, zero_lane, carry)
    lax.fori_loop(0, _CHUNK, zero_rows, 0)

    def zero_stripe(k, carry):
      pltpu.sync_copy(rows_v, acc_sh.at[pl.ds(r0 + k * _CHUNK, _CHUNK)])
      return carry
    lax.fori_loop(0, _STRIPE // _CHUNK, zero_stripe, 0)
    if feat_in_spmem:
      pltpu.sync_copy(feat_hbm.at[pl.ds(r0, _STRIPE)],
                      feat_sh.at[pl.ds(r0, _STRIPE)])
    if with_count:
      def zero_cnt(i, carry):
        cnt_v[pl.ds(i * 16, 16)] = jnp.zeros((16,), jnp.float32)
        return carry
      lax.fori_loop(0, _NPAD // 16, zero_cnt, 0)
    plsc.subcore_barrier()

    def body(c, carry):
      feat_ref = feat_sh if feat_in_spmem else feat_hbm
      gat = pltpu.async_copy(feat_ref.at[src_v.at[c]], rows_v, sem)
      if with_count:
        for j in range(_VPC):
          idx = dst_v[c, pl.ds(j * 16, 16)]
          cnts, last = plsc.scan_count(idx)
          plsc.addupdate_scatter(cnt_v, [idx], cnts.astype(jnp.float32),
                                 mask=last)
      gat.wait()
      pltpu.sync_copy(rows_v, acc_sh.at[dst_v.at[c]], add=True)
      return carry

    lax.fori_loop(0, _CPT, body, 0)

    if with_count:
      pltpu.sync_copy(cnt_v, cnt_hbm.at[wid])
    plsc.subcore_barrier()
    pltpu.sync_copy(acc_sh.at[pl.ds(r0, _STRIPE)],
                    out_hbm.at[cid, pl.ds(r0, _STRIPE)])

  return agg


_agg_cache = {}


def _agg(key, D, with_count, *args):
  if key not in _agg_cache:
    _agg_cache[key] = _make_sc_agg(D, with_count, feat_in_spmem=True)
  return _agg_cache[key](*args)


_RB = 640
_GRID = _NPAD // _RB


def _tc_mid_body(acca0_ref, acca1_ref, accb0_ref, accb1_ref, cnt_ref, h_ref,
                 w1lt_ref, b1_ref, w1r_ref, w2l_ref, w2r_ref, b2_ref,
                 z_ref, r2_ref, inv_ref):
  acc_a = acca0_ref[...] + acca1_ref[...]
  acc_b = accb0_ref[...] + accb1_ref[...]
  ones = jnp.full((_NW, 1), 1.0, dtype=jnp.float32)
  cnt = lax.dot_general(cnt_ref[...], ones, (((0,), (0,)), ((), ())),
                        preferred_element_type=jnp.float32)
  inv = 1.0 / jnp.maximum(cnt, 1.0)
  mm = (jnp.dot(acc_a, w1lt_ref[...][:_DH], preferred_element_type=jnp.float32)
        + jnp.dot(acc_b, w1lt_ref[...][_DH:],
                  preferred_element_type=jnp.float32))
  x1 = (mm * inv + b1_ref[...]
        + jnp.dot(h_ref[...], w1r_ref[...], preferred_element_type=jnp.float32))
  x1 = jnp.maximum(x1, 0.0)
  z_ref[...] = jnp.dot(x1, w2l_ref[...], preferred_element_type=jnp.float32)
  r2_ref[...] = (jnp.dot(x1, w2r_ref[...], preferred_element_type=jnp.float32)
                 + b2_ref[...])
  inv_ref[...] = jnp.broadcast_to(inv, (_RB, _D2))


def _tc_mid(acc_a0, acc_a1, acc_b0, acc_b1, cnt_parts, h_pad, w1l_t, b1r,
            w1r_t, w2l_pad, w2r_pad, b2r):
  blk = lambda r, c: pl.BlockSpec((r, c), lambda i: (i, 0))
  full = lambda r, c: pl.BlockSpec((r, c), lambda i: (0, 0))
  return pl.pallas_call(
      _tc_mid_body,
      grid=(_GRID,),
      in_specs=[
          blk(_RB, _DH), blk(_RB, _DH), blk(_RB, _DH), blk(_RB, _DH),
          pl.BlockSpec((_NW, _RB), lambda i: (0, i)),
          blk(_RB, _D),
          full(_D, _D), full(1, _D), full(_D, _D),
          full(_D, _D2), full(_D, _D2), full(1, _D2),
      ],
      out_specs=[blk(_RB, _D2), blk(_RB, _D2), blk(_RB, _D2)],
      out_shape=[
          jax.ShapeDtypeStruct((_NPAD, _D2), jnp.float32),
          jax.ShapeDtypeStruct((_NPAD, _D2), jnp.float32),
          jax.ShapeDtypeStruct((_NPAD, _D2), jnp.float32),
      ],
  )(acc_a0, acc_a1, acc_b0, acc_b1, cnt_parts, h_pad, w1l_t, b1r, w1r_t,
    w2l_pad, w2r_pad, b2r)


def _tc_out_body(acc2a_ref, acc2b_ref, inv_ref, r2_ref, out_ref):
  s = (acc2a_ref[...] + acc2b_ref[...]) * inv_ref[...] + r2_ref[...]
  lane = lax.broadcasted_iota(jnp.int32, (_RB, _D2), 1)
  sm = jnp.where(lane < _NCLS, s, -jnp.inf)
  m = jnp.max(sm, axis=1, keepdims=True)
  e = jnp.where(lane < _NCLS, jnp.exp(sm - m), 0.0)
  out_ref[...] = s - (jnp.log(jnp.sum(e, axis=1, keepdims=True)) + m)


def _tc_out(acc2_a, acc2_b, inv48, r2b):
  blk = pl.BlockSpec((_RB, _D2), lambda i: (i, 0))
  return pl.pallas_call(
      _tc_out_body,
      grid=(_GRID,),
      in_specs=[blk, blk, blk, blk],
      out_specs=blk,
      out_shape=jax.ShapeDtypeStruct((_NPAD, _D2), jnp.float32),
  )(acc2_a, acc2_b, inv48, r2b)


def kernel(h, edge_index, W1l, b1, W1r, W2l, b2, W2r):
  h = h.astype(jnp.float32)
  src = edge_index[0].astype(jnp.int32)
  dst = edge_index[1].astype(jnp.int32)

  pad = _EPAD - _E
  src_p = jnp.concatenate([src, jnp.zeros((pad,), jnp.int32)])
  dst_p = jnp.concatenate([dst, jnp.full((pad,), _N, jnp.int32)])
  src_p = src_p.reshape(_NW, _CPT, _CHUNK)
  dst_p = dst_p.reshape(_NW, _CPT, _CHUNK)

  h_pad = jnp.zeros((_NPAD, _D), jnp.float32).at[:_N].set(h)

  w1l_t = W1l.T
  w1r_t = W1r.T
  w2l_pad = jnp.zeros((_D, _D2), jnp.float32).at[:, :_NCLS].set(W2l.T)
  w2r_pad = jnp.zeros((_D, _D2), jnp.float32).at[:, :_NCLS].set(W2r.T)
  b1r = b1.reshape(1, _D)
  b2r = jnp.zeros((1, _D2), jnp.float32).at[0, :_NCLS].set(b2)

  h_a = h_pad[:, :_DH].copy()
  h_b = h_pad[:, _DH:].copy()
  acc1a, cnt_parts = _agg(1, _DH, True, h_a, src_p, dst_p)
  acc1b = _agg(2, _DH, False, h_b, src_p, dst_p)[0]
  z, r2b, inv48 = _tc_mid(acc1a[0], acc1a[1], acc1b[0], acc1b[1], cnt_parts,
                          h_pad, w1l_t, b1r, w1r_t, w2l_pad, w2r_pad, b2r)
  acc2 = _agg(3, _D2, False, z, src_p, dst_p)[0]
  out = _tc_out(acc2[0], acc2[1], inv48, r2b)
  return out[:_N, :_NCLS]


# speedup vs baseline: 8.9195x; 1.0425x over previous
import functools

import jax
import jax.numpy as jnp
from jax import lax
from jax.experimental import pallas as pl
from jax.experimental.pallas import tpu as pltpu
from jax.experimental.pallas import tpu_sc as plsc

_N = 10000
_E = 320000
_D = 128
_NCLS = 47

_NPAD = 10240
_D2 = 48
_DH = 64

_NC = 2
_NS = 16
_NW = _NC * _NS
_CHUNK = 128
_CPT = 80
_EPAD = _NW * _CPT * _CHUNK
_STRIPE = _NPAD // _NS
_VPC = _CHUNK // 16
_SCW = 4


def _make_sc_agg(D, with_count, feat_in_spmem=False):
  mesh = plsc.VectorSubcoreMesh(core_axis_name="c", subcore_axis_name="s",
                                num_cores=_NC, num_subcores=_NS)
  out_type = [jax.ShapeDtypeStruct((_NC, _NPAD, D), jnp.float32)]
  scratch = [
      pltpu.VMEM((_CPT, _CHUNK), jnp.int32),
      pltpu.VMEM((_CPT, _CHUNK), jnp.int32),
      pltpu.VMEM((_CHUNK, D), jnp.float32),
      pltpu.VMEM_SHARED((_NPAD, D), jnp.float32),
      pltpu.SemaphoreType.DMA,
  ]
  if with_count:
    out_type.append(jax.ShapeDtypeStruct((_NW, _NPAD), jnp.float32))
    scratch.append(pltpu.VMEM((_NPAD,), jnp.float32))
  if feat_in_spmem:
    scratch.append(pltpu.VMEM_SHARED((_NPAD, D), jnp.float32))

  @functools.partial(
      pl.kernel,
      out_type=out_type,
      mesh=mesh,
      compiler_params=pltpu.CompilerParams(use_tc_tiling_on_sc=False,
                                           needs_layout_passes=False),
      scratch_types=scratch,
  )
  def agg(feat_hbm, srcs_hbm, dsts_hbm, *rest):
    feat_sh = rest[-1] if feat_in_spmem else None
    if feat_in_spmem:
      rest = rest[:-1]
    if with_count:
      out_hbm, cnt_hbm, src_v, dst_v, rows_v, acc_sh, sem, cnt_v = rest
    else:
      out_hbm, src_v, dst_v, rows_v, acc_sh, sem = rest
    cid = lax.axis_index("c")
    sid = lax.axis_index("s")
    wid = sid * _NC + cid
    r0 = sid * _STRIPE

    pltpu.sync_copy(srcs_hbm.at[wid], src_v)
    pltpu.sync_copy(dsts_hbm.at[wid], dst_v)

    def zero_rows(i, carry):
      def zero_lane(j, carry2):
        rows_v[i, pl.ds(j * 16, 16)] = jnp.zeros((16,), jnp.float32)
        return carry2
      return lax.fori_loop(0, D // 16, zero_lane, carry)
    lax.fori_loop(0, _CHUNK, zero_rows, 0)

    def zero_stripe(k, carry):
      pltpu.sync_copy(rows_v, acc_sh.at[pl.ds(r0 + k * _CHUNK, _CHUNK)])
      return carry
    lax.fori_loop(0, _STRIPE // _CHUNK, zero_stripe, 0)
    if feat_in_spmem:
      pltpu.sync_copy(feat_hbm.at[pl.ds(r0, _STRIPE)],
                      feat_sh.at[pl.ds(r0, _STRIPE)])
    if with_count:
      def zero_cnt(i, carry):
        cnt_v[pl.ds(i * 16, 16)] = jnp.zeros((16,), jnp.float32)
        return carry
      lax.fori_loop(0, _NPAD // 16, zero_cnt, 0)
    plsc.subcore_barrier()

    def body(c, carry):
      feat_ref = feat_sh if feat_in_spmem else feat_hbm
      gat = pltpu.async_copy(feat_ref.at[src_v.at[c]], rows_v, sem)
      if with_count:
        for j in range(_VPC):
          idx = dst_v[c, pl.ds(j * 16, 16)]
          cnts, last = plsc.scan_count(idx)
          plsc.addupdate_scatter(cnt_v, [idx], cnts.astype(jnp.float32),
                                 mask=last)
      gat.wait()
      pltpu.sync_copy(rows_v, acc_sh.at[dst_v.at[c]], add=True)
      return carry

    lax.fori_loop(0, _CPT, body, 0)

    if with_count:
      pltpu.sync_copy(cnt_v, cnt_hbm.at[wid])
    plsc.subcore_barrier()
    pltpu.sync_copy(acc_sh.at[pl.ds(r0, _STRIPE)],
                    out_hbm.at[cid, pl.ds(r0, _STRIPE)])

  return agg


_agg_cache = {}


def _agg(key, D, with_count, *args):
  if key not in _agg_cache:
    _agg_cache[key] = _make_sc_agg(D, with_count, feat_in_spmem=True)
  return _agg_cache[key](*args)


_RB = 640
_GRID = _NPAD // _RB


def _tc_mid_body(acca0_ref, acca1_ref, accb0_ref, accb1_ref, cnt_ref, h_ref,
                 w1lt_ref, b1_ref, w1r_ref, w2l_ref, w2r_ref, b2_ref,
                 z_ref, r2_ref, inv_ref):
  acc_a = acca0_ref[0] + acca1_ref[0]
  acc_b = accb0_ref[0] + accb1_ref[0]
  ones = jnp.full((_NW, 1), 1.0, dtype=jnp.float32)
  cnt = lax.dot_general(cnt_ref[...], ones, (((0,), (0,)), ((), ())),
                        preferred_element_type=jnp.float32)
  inv = 1.0 / jnp.maximum(cnt, 1.0)
  mm = (jnp.dot(acc_a, w1lt_ref[...][:_DH], preferred_element_type=jnp.float32)
        + jnp.dot(acc_b, w1lt_ref[...][_DH:],
                  preferred_element_type=jnp.float32))
  x1 = (mm * inv + b1_ref[...]
        + jnp.dot(h_ref[...], w1r_ref[...], preferred_element_type=jnp.float32))
  x1 = jnp.maximum(x1, 0.0)
  z_ref[...] = jnp.dot(x1, w2l_ref[...], preferred_element_type=jnp.float32)
  r2_ref[...] = (jnp.dot(x1, w2r_ref[...], preferred_element_type=jnp.float32)
                 + b2_ref[...])
  inv_ref[...] = jnp.broadcast_to(inv, (_RB, _D2))


def _tc_mid(acc_a, acc_b, cnt_parts, h_pad, w1l_t, b1r,
            w1r_t, w2l_pad, w2r_pad, b2r):
  blk = lambda r, c: pl.BlockSpec((r, c), lambda i: (i, 0))
  full = lambda r, c: pl.BlockSpec((r, c), lambda i: (0, 0))
  return pl.pallas_call(
      _tc_mid_body,
      grid=(_GRID,),
      in_specs=[
          pl.BlockSpec((1, _RB, _DH), lambda i: (0, i, 0)),
          pl.BlockSpec((1, _RB, _DH), lambda i: (1, i, 0)),
          pl.BlockSpec((1, _RB, _DH), lambda i: (0, i, 0)),
          pl.BlockSpec((1, _RB, _DH), lambda i: (1, i, 0)),
          pl.BlockSpec((_NW, _RB), lambda i: (0, i)),
          blk(_RB, _D),
          full(_D, _D), full(1, _D), full(_D, _D),
          full(_D, _D2), full(_D, _D2), full(1, _D2),
      ],
      out_specs=[blk(_RB, _D2), blk(_RB, _D2), blk(_RB, _D2)],
      out_shape=[
          jax.ShapeDtypeStruct((_NPAD, _D2), jnp.float32),
          jax.ShapeDtypeStruct((_NPAD, _D2), jnp.float32),
          jax.ShapeDtypeStruct((_NPAD, _D2), jnp.float32),
      ],
  )(acc_a, acc_a, acc_b, acc_b, cnt_parts, h_pad, w1l_t, b1r, w1r_t,
    w2l_pad, w2r_pad, b2r)


def _tc_out_body(acc2a_ref, acc2b_ref, inv_ref, r2_ref, out_ref):
  s = (acc2a_ref[0] + acc2b_ref[0]) * inv_ref[...] + r2_ref[...]
  lane = lax.broadcasted_iota(jnp.int32, (_RB, _D2), 1)
  sm = jnp.where(lane < _NCLS, s, -jnp.inf)
  m = jnp.max(sm, axis=1, keepdims=True)
  e = jnp.where(lane < _NCLS, jnp.exp(sm - m), 0.0)
  out_ref[...] = s - (jnp.log(jnp.sum(e, axis=1, keepdims=True)) + m)


def _tc_out(acc2, inv48, r2b):
  blk = pl.BlockSpec((_RB, _D2), lambda i: (i, 0))
  return pl.pallas_call(
      _tc_out_body,
      grid=(_GRID,),
      in_specs=[pl.BlockSpec((1, _RB, _D2), lambda i: (0, i, 0)),
                pl.BlockSpec((1, _RB, _D2), lambda i: (1, i, 0)),
                blk, blk],
      out_specs=blk,
      out_shape=jax.ShapeDtypeStruct((_NPAD, _D2), jnp.float32),
  )(acc2, acc2, inv48, r2b)


def kernel(h, edge_index, W1l, b1, W1r, W2l, b2, W2r):
  h = h.astype(jnp.float32)
  src = edge_index[0].astype(jnp.int32)
  dst = edge_index[1].astype(jnp.int32)

  pad = _EPAD - _E
  src_p = jnp.concatenate([src, jnp.zeros((pad,), jnp.int32)])
  dst_p = jnp.concatenate([dst, jnp.full((pad,), _N, jnp.int32)])
  src_p = src_p.reshape(_NW, _CPT, _CHUNK)
  dst_p = dst_p.reshape(_NW, _CPT, _CHUNK)

  h_pad = jnp.zeros((_NPAD, _D), jnp.float32).at[:_N].set(h)

  w1l_t = W1l.T
  w1r_t = W1r.T
  w2l_pad = jnp.zeros((_D, _D2), jnp.float32).at[:, :_NCLS].set(W2l.T)
  w2r_pad = jnp.zeros((_D, _D2), jnp.float32).at[:, :_NCLS].set(W2r.T)
  b1r = b1.reshape(1, _D)
  b2r = jnp.zeros((1, _D2), jnp.float32).at[0, :_NCLS].set(b2)

  h_a = h_pad[:, :_DH].copy()
  h_b = h_pad[:, _DH:].copy()
  acc1a, cnt_parts = _agg(1, _DH, True, h_a, src_p, dst_p)
  acc1b = _agg(2, _DH, False, h_b, src_p, dst_p)[0]
  z, r2b, inv48 = _tc_mid(acc1a, acc1b, cnt_parts,
                          h_pad, w1l_t, b1r, w1r_t, w2l_pad, w2r_pad, b2r)
  acc2 = _agg(3, _D2, False, z, src_p, dst_p)[0]
  out = _tc_out(acc2, inv48, r2b)
  return out[:_N, :_NCLS]
